# baseline (speedup 1.0000x reference)
import functools

import jax
import jax.numpy as jnp
from jax import lax
from jax.experimental import pallas as pl
from jax.experimental.pallas import tpu as pltpu
from jax.experimental.pallas import tpu_sc as plsc

NB_VECS = 1000000
NB_DIMS = 64
BATCH = 16384
PAIR = 2

NC = 2
NS = 16
NW = NC * NS
LANES = 16

ROWS_PER_W = BATCH * PAIR // NW
PAIRS_PER_W = BATCH // NW
GCHUNK = 128
NCHUNK = ROWS_PER_W // GCHUNK
DCHUNK = NB_DIMS // LANES

PANEL = 256
SKEW = PANEL + 1
NPANEL = NB_VECS // PANEL
TAIL = NB_VECS - NPANEL * PANEL
BASE_PANELS = NPANEL // NW
EXTRA_PANELS = NPANEL - BASE_PANELS * NW
PACKED = NB_VECS // 2

_params = pltpu.CompilerParams(use_tc_tiling_on_sc=True,
                               needs_layout_passes=False)


TCBLK = 512
TCGRID = -(-NB_VECS // TCBLK)


def _transpose_table(wt):
    def body(x_ref, y_ref):
        t = x_ref[...].T.reshape(TCBLK // 2, 2, NB_DIMS)
        y_ref[...] = jnp.concatenate([t[:, 0, :], t[:, 1, :]], axis=1)

    return pl.pallas_call(
        body,
        grid=(TCGRID,),
        in_specs=[pl.BlockSpec((NB_DIMS, TCBLK), lambda j: (0, j))],
        out_specs=pl.BlockSpec((TCBLK // 2, 2 * NB_DIMS),
                               lambda j: (j, 0)),
        out_shape=jax.ShapeDtypeStruct((PACKED, 2 * NB_DIMS),
                                       jnp.float32),
    )(wt)


def _gather_dot(vii_r, par_r, sc):
    mesh = plsc.VectorSubcoreMesh(core_axis_name="c", subcore_axis_name="s")

    @functools.partial(
        pl.kernel,
        out_type=jax.ShapeDtypeStruct((BATCH,), jnp.float32),
        mesh=mesh,
        compiler_params=_params,
        scratch_types=[
            pltpu.VMEM((NCHUNK, GCHUNK), jnp.int32),
            pltpu.VMEM((ROWS_PER_W // 2,), jnp.int32),
            pltpu.VMEM((2, GCHUNK, 2 * NB_DIMS), jnp.float32),
            pltpu.VMEM((PAIRS_PER_W,), jnp.float32),
            pltpu.SemaphoreType.DMA,
            pltpu.SemaphoreType.DMA,
        ],
    )
    def k2(vii_hbm, par_hbm, sc_hbm, out_hbm, idx_v, par_v, rows_v,
           out_v, sem0, sem1):
        wid = lax.axis_index("c") * NS + lax.axis_index("s")
        sems = (sem0, sem1)

        pltpu.sync_copy(vii_hbm.at[wid], idx_v)
        pltpu.sync_copy(par_hbm.at[wid], par_v)

        lane = lax.iota(jnp.int32, LANES)
        bfly = [lane ^ (1 << s) for s in range(4)]

        def hsum(v):
            for idx in bfly:
                v = v + jnp.take(v, idx)
            return v

        def fire(j):
            pltpu.async_copy(
                sc_hbm.at[idx_v.at[j]],
                rows_v.at[j % 2],
                sems[j % 2],
            )

        def drain(j):
            pltpu.make_async_copy(
                sc_hbm.at[pl.ds(0, GCHUNK)],
                rows_v.at[j % 2],
                sems[j % 2],
            ).wait()

        def compute(j):
            buf = rows_v.at[j % 2]

            def group_body(g, _):
                pav = par_v[pl.ds(j * (GCHUNK // 2) + g * LANES, LANES)]
                res = jnp.zeros((LANES,), jnp.float32)
                for jj in range(LANES):
                    i = g * LANES + jj
                    pbits = pav[jj]
                    pa = (pbits & 1) != 0
                    pb = (pbits & 2) != 0
                    acc = None
                    for kk in range(DCHUNK):
                        a_lo = buf[2 * i, pl.ds(kk * LANES, LANES)]
                        a_hi = buf[2 * i,
                                   pl.ds(NB_DIMS + kk * LANES, LANES)]
                        b_lo = buf[2 * i + 1, pl.ds(kk * LANES, LANES)]
                        b_hi = buf[2 * i + 1,
                                   pl.ds(NB_DIMS + kk * LANES, LANES)]
                        va = jnp.where(pa, a_hi, a_lo)
                        vb = jnp.where(pb, b_hi, b_lo)
                        p = va * vb
                        acc = p if acc is None else acc + p
                    res = jnp.where(lane == jj, hsum(acc), res)
                out_v[pl.ds(j * (GCHUNK // 2) + g * LANES, LANES)] = res
                return 0

            lax.fori_loop(0, GCHUNK // 2 // LANES, group_body, 0)

        fire(0)
        for j in range(1, NCHUNK):
            fire(j)
            drain(j - 1)
            compute(j - 1)
        drain(NCHUNK - 1)
        compute(NCHUNK - 1)

        pltpu.sync_copy(out_v, out_hbm.at[pl.ds(wid * PAIRS_PER_W,
                                                PAIRS_PER_W)])

    return k2(vii_r, par_r, sc)


def kernel(vii, W):
    vii32 = vii.astype(jnp.int32)
    flat = vii32.reshape(-1)
    packed = (flat >> 1).reshape(NW, NCHUNK, GCHUNK)
    par = (vii32[:, 0] & 1) | ((vii32[:, 1] & 1) << 1)
    par_r = par.reshape(NW, PAIRS_PER_W)
    wt = W.T
    sc = _transpose_table(wt)
    return _gather_dot(packed, par_r, sc)


# speedup vs baseline: 3.4023x; 3.4023x over previous
import functools

import jax
import jax.numpy as jnp
from jax import lax
from jax.experimental import pallas as pl
from jax.experimental.pallas import tpu as pltpu
from jax.experimental.pallas import tpu_sc as plsc

NB_VECS = 1000000
NB_DIMS = 64
BATCH = 16384
PAIR = 2

NC = 2
NS = 16
NW = NC * NS
LANES = 16

ROWS_PER_W = BATCH * PAIR // NW
PAIRS_PER_W = BATCH // NW
GCHUNK = 128
NCHUNK = ROWS_PER_W // GCHUNK
DCHUNK = NB_DIMS // LANES


def _sgns(vii_r, W):
    mesh = plsc.VectorSubcoreMesh(core_axis_name="c", subcore_axis_name="s")

    @functools.partial(
        pl.kernel,
        out_type=jax.ShapeDtypeStruct((BATCH,), jnp.float32),
        mesh=mesh,
        compiler_params=pltpu.CompilerParams(use_tc_tiling_on_sc=True),
        scratch_types=[
            pltpu.VMEM((ROWS_PER_W,), jnp.int32),
            pltpu.VMEM((2, GCHUNK, NB_DIMS), jnp.float32),
            pltpu.VMEM((PAIRS_PER_W,), jnp.float32),
            pltpu.SemaphoreType.DMA,
            pltpu.SemaphoreType.DMA,
        ],
    )
    def k(vii_hbm, w_hbm, out_hbm, idx_v, rows_v, out_v, sem0, sem1):
        sems = (sem0, sem1)
        wid = lax.axis_index("c") * NS + lax.axis_index("s")

        pltpu.sync_copy(vii_hbm.at[wid], idx_v)

        lane = lax.iota(jnp.int32, LANES)
        bfly = [lane ^ (1 << s) for s in range(4)]

        def hsum(v):
            for idx in bfly:
                v = v + jnp.take(v, idx)
            return v

        def fire(j):
            buf = rows_v.at[j % 2]

            def body(g, _):
                iv = idx_v[pl.ds(j * GCHUNK + g * LANES, LANES)]
                for kk in range(LANES):
                    pltpu.async_copy(
                        w_hbm.at[pl.ds(iv[kk], 1), :],
                        buf.at[pl.ds(g * LANES + kk, 1), :],
                        sems[j % 2],
                    )
                return 0

            lax.fori_loop(0, GCHUNK // LANES, body, 0)

        def drain(j):
            pltpu.make_async_copy(
                w_hbm.at[pl.ds(0, GCHUNK), :],
                rows_v.at[j % 2],
                sems[j % 2],
            ).wait()

        def compute(j):
            buf = rows_v.at[j % 2]

            def group_body(g, _):
                res = jnp.zeros((LANES,), jnp.float32)
                for jj in range(LANES):
                    i = g * LANES + jj
                    acc = (buf[2 * i, pl.ds(0, LANES)]
                           * buf[2 * i + 1, pl.ds(0, LANES)])
                    for kk in range(1, DCHUNK):
                        acc = acc + (buf[2 * i, pl.ds(kk * LANES, LANES)]
                                     * buf[2 * i + 1, pl.ds(kk * LANES,
                                                            LANES)])
                    res = jnp.where(lane == jj, hsum(acc), res)
                out_v[pl.ds(j * (GCHUNK // 2) + g * LANES, LANES)] = res
                return 0

            lax.fori_loop(0, GCHUNK // 2 // LANES, group_body, 0)

        fire(0)
        for j in range(1, NCHUNK):
            fire(j)
            drain(j - 1)
            compute(j - 1)
        drain(NCHUNK - 1)
        compute(NCHUNK - 1)

        pltpu.sync_copy(out_v, out_hbm.at[pl.ds(wid * PAIRS_PER_W,
                                                PAIRS_PER_W)])

    return k(vii_r, W)


def kernel(vii, W):
    vii_r = vii.astype(jnp.int32).reshape(NW, ROWS_PER_W)
    return _sgns(vii_r, W)


# speedup vs baseline: 3.5106x; 1.0319x over previous
import functools

import jax
import jax.numpy as jnp
from jax import lax
from jax.experimental import pallas as pl
from jax.experimental.pallas import tpu as pltpu
from jax.experimental.pallas import tpu_sc as plsc

NB_VECS = 1000000
NB_DIMS = 64
BATCH = 16384
PAIR = 2

NC = 2
NS = 16
NW = NC * NS
LANES = 16

ROWS_PER_W = BATCH * PAIR // NW
PAIRS_PER_W = BATCH // NW
GCHUNK = 128
NCHUNK = ROWS_PER_W // GCHUNK
DCHUNK = NB_DIMS // LANES


def _sgns(vii_r, W):
    mesh = plsc.VectorSubcoreMesh(core_axis_name="c", subcore_axis_name="s")

    @functools.partial(
        pl.kernel,
        out_type=jax.ShapeDtypeStruct((BATCH,), jnp.float32),
        mesh=mesh,
        compiler_params=pltpu.CompilerParams(use_tc_tiling_on_sc=True),
        scratch_types=[
            pltpu.VMEM((PAIRS_PER_W,), jnp.int32),
            pltpu.VMEM((PAIRS_PER_W,), jnp.int32),
            pltpu.VMEM((2, GCHUNK, NB_DIMS), jnp.float32),
            pltpu.VMEM((PAIRS_PER_W,), jnp.float32),
            pltpu.SemaphoreType.DMA,
            pltpu.SemaphoreType.DMA,
        ],
    )
    def k(vii_hbm, w_hbm, out_hbm, ia_v, ib_v, rows_v, out_v, sem0,
          sem1):
        sems = (sem0, sem1)
        wid = lax.axis_index("c") * NS + lax.axis_index("s")

        base = pl.multiple_of(wid * PAIRS_PER_W, PAIRS_PER_W)
        pltpu.sync_copy(vii_hbm.at[0, pl.ds(base, PAIRS_PER_W)], ia_v)
        pltpu.sync_copy(vii_hbm.at[1, pl.ds(base, PAIRS_PER_W)], ib_v)

        lane = lax.iota(jnp.int32, LANES)
        bfly = [lane ^ (1 << s) for s in range(4)]

        def hsum(v):
            for idx in bfly:
                v = v + jnp.take(v, idx)
            return v

        def fire(j):
            buf = rows_v.at[j % 2]

            def body(g, _):
                off = j * (GCHUNK // 2) + g * LANES
                iva = ia_v[pl.ds(off, LANES)]
                ivb = ib_v[pl.ds(off, LANES)]
                for kk in range(LANES):
                    pltpu.async_copy(
                        w_hbm.at[pl.ds(iva[kk], 1), :],
                        buf.at[pl.ds(2 * (g * LANES + kk), 1), :],
                        sems[j % 2],
                    )
                    pltpu.async_copy(
                        w_hbm.at[pl.ds(ivb[kk], 1), :],
                        buf.at[pl.ds(2 * (g * LANES + kk) + 1, 1), :],
                        sems[j % 2],
                    )
                return 0

            lax.fori_loop(0, GCHUNK // 2 // LANES, body, 0)

        def drain(j):
            pltpu.make_async_copy(
                w_hbm.at[pl.ds(0, GCHUNK), :],
                rows_v.at[j % 2],
                sems[j % 2],
            ).wait()

        def compute(j):
            buf = rows_v.at[j % 2]

            def group_body(g, _):
                res = jnp.zeros((LANES,), jnp.float32)
                for jj in range(LANES):
                    i = g * LANES + jj
                    acc = (buf[2 * i, pl.ds(0, LANES)]
                           * buf[2 * i + 1, pl.ds(0, LANES)])
                    for kk in range(1, DCHUNK):
                        acc = acc + (buf[2 * i, pl.ds(kk * LANES, LANES)]
                                     * buf[2 * i + 1, pl.ds(kk * LANES,
                                                            LANES)])
                    res = jnp.where(lane == jj, hsum(acc), res)
                out_v[pl.ds(j * (GCHUNK // 2) + g * LANES, LANES)] = res
                return 0

            lax.fori_loop(0, GCHUNK // 2 // LANES, group_body, 0)

        fire(0)
        for j in range(1, NCHUNK):
            fire(j)
            drain(j - 1)
            compute(j - 1)
        drain(NCHUNK - 1)
        compute(NCHUNK - 1)

        pltpu.sync_copy(out_v, out_hbm.at[pl.ds(wid * PAIRS_PER_W,
                                                PAIRS_PER_W)])

    return k(vii_r, W)


def kernel(vii, W):
    vii_t = vii.astype(jnp.int32).T
    return _sgns(vii_t, W)
